# Initial kernel scaffold; baseline (speedup 1.0000x reference)
import jax
import jax.numpy as jnp
from jax import lax
from jax.experimental import pallas as pl
from jax.experimental.pallas import tpu as pltpu
from jax.experimental.pallas import tpu_sc as plsc

B = 16384
F = 26
V = 100000
D = 16

NC = 2
NS = 16
NW = NC * NS
EPW = B // NW
CE = 128
NCHUNK = EPW // CE


def _body(tab_hbm, idx_hbm, out_hbm, idx_v, rows_v, out_v, sem):
  c = lax.axis_index("c")
  s = lax.axis_index("s")
  wid = s * NC + c

  pltpu.sync_copy(idx_hbm.at[wid], idx_v)

  for ch in range(NCHUNK):
    descs = []
    for f in range(F):
      d = pltpu.async_copy(
          tab_hbm.at[idx_v.at[f, pl.ds(ch * CE, CE)]],
          rows_v.at[f],
          sem,
      )
      descs.append(d)
    for d in descs:
      d.wait()

    def red(e, carry):
      acc = rows_v[0, e, :]
      for f in range(1, F):
        acc = acc + rows_v[f, e, :]
      out_v[e, :] = acc
      return carry

    lax.fori_loop(0, CE, red, 0)
    pltpu.sync_copy(out_v, out_hbm.at[pl.ds(wid * EPW + ch * CE, CE)])


@jax.jit
def _embed_sum(tables2, idx3):
  mesh = plsc.VectorSubcoreMesh(core_axis_name="c", subcore_axis_name="s")
  return pl.kernel(
      _body,
      out_type=jax.ShapeDtypeStruct((B, D), jnp.float32),
      mesh=mesh,
      scratch_types=[
          pltpu.VMEM((F, EPW), jnp.int32),
          pltpu.VMEM((F, CE, D), jnp.float32),
          pltpu.VMEM((CE, D), jnp.float32),
          pltpu.SemaphoreType.DMA,
      ],
  )(tables2, idx3)


def kernel(x, tables):
  tables2 = tables.reshape(F * V, D)
  offs = (jnp.arange(F, dtype=jnp.int32) * V)[None, :]
  idx = x + offs
  idx3 = idx.reshape(NW, EPW, F).transpose(0, 2, 1)
  return _embed_sum(tables2, idx3)


# speedup vs baseline: 1.0389x; 1.0389x over previous
import jax
import jax.numpy as jnp
from jax import lax
from jax.experimental import pallas as pl
from jax.experimental.pallas import tpu as pltpu
from jax.experimental.pallas import tpu_sc as plsc

B = 16384
F = 26
V = 100000
D = 16

NC = 2
NS = 16
NW = NC * NS
EPW = B // NW
CE = 128
NCHUNK = EPW // CE


def _body(tab_hbm, idx_hbm, out_hbm, idx_v, rows_v, out_v, sem):
  c = lax.axis_index("c")
  s = lax.axis_index("s")
  wid = s * NC + c

  pltpu.sync_copy(idx_hbm.at[wid], idx_v)

  for ch in range(NCHUNK):
    descs = []
    for f in range(F):
      d = pltpu.async_copy(
          tab_hbm.at[idx_v.at[f, pl.ds(ch * CE, CE)]],
          rows_v.at[f],
          sem,
      )
      descs.append(d)
    for d in descs:
      d.wait()

    def red(e, carry):
      acc = rows_v[0, e, :]
      for f in range(1, F):
        acc = acc + rows_v[f, e, :]
      out_v[e, :] = acc
      return carry

    lax.fori_loop(0, CE, red, 0)
    pltpu.sync_copy(out_v, out_hbm.at[pl.ds(wid * EPW + ch * CE, CE)])


@jax.jit
def _embed_sum(tables2, idx3):
  mesh = plsc.VectorSubcoreMesh(core_axis_name="c", subcore_axis_name="s")
  return pl.kernel(
      _body,
      out_type=jax.ShapeDtypeStruct((B, D), jnp.float32),
      mesh=mesh,
      scratch_types=[
          pltpu.VMEM((F, EPW), jnp.int32),
          pltpu.VMEM((F, CE, D), jnp.float32),
          pltpu.VMEM((CE, D), jnp.float32),
          pltpu.SemaphoreType.DMA,
      ],
      compiler_params=pltpu.CompilerParams(use_tc_tiling_on_sc=False),
  )(tables2, idx3)


def kernel(x, tables):
  tables2 = tables.reshape(F * V, D)
  offs = (jnp.arange(F, dtype=jnp.int32) * V)[None, :]
  idx = x + offs
  idx3 = idx.reshape(NW, EPW, F).transpose(0, 2, 1)
  return _embed_sum(tables2, idx3)
